# baseline (speedup 1.0000x reference)
import jax
import jax.numpy as jnp
from jax.experimental import pallas as pl
from jax.experimental.pallas import tpu as pltpu

_B, _N, _H = 1024, 50, 128
_ACT = 19
_BB = 32
_R = _BB * _N
_NP = 64
_P = _BB * _NP


def _fused(obs_ref, rnn_ref, masks_ref, W_obs_ref, b_obs_ref, W_ih_ref,
           W_hh_ref, b_gates_ref, W_me_ref, b_me_ref, W_g1_ref, A1_ref,
           W_g2_ref, A2_ref, W_md_ref, b_md_ref, W_act_ref, b_act_ref,
           out_ref):
    f32 = jnp.float32

    x = jnp.tanh(jnp.dot(obs_ref[...], W_obs_ref[...],
                         preferred_element_type=f32) + b_obs_ref[...])
    m = masks_ref[...]
    h0 = rnn_ref[:, :_H] * m
    c0 = rnn_ref[:, _H:] * m
    gates = (jnp.dot(x, W_ih_ref[...], preferred_element_type=f32)
             + jnp.dot(h0, W_hh_ref[...], preferred_element_type=f32)
             + b_gates_ref[...])
    ig = jax.nn.sigmoid(gates[:, :_H])
    fg = jax.nn.sigmoid(gates[:, _H:2 * _H])
    gg = jnp.tanh(gates[:, 2 * _H:3 * _H])
    og = jax.nn.sigmoid(gates[:, 3 * _H:])
    c = fg * c0 + ig * gg
    h = og * jnp.tanh(c)

    comm = jnp.dot(h, W_me_ref[...], preferred_element_type=f32) + b_me_ref[...]

    zpad = jnp.zeros((_NP - _N, _H), f32)
    pieces = []
    for k in range(_BB):
        pieces.append(comm[k * _N:(k + 1) * _N, :])
        pieces.append(zpad)
    cp = jnp.concatenate(pieces, axis=0)

    lane = jax.lax.broadcasted_iota(jnp.int32, (1, 1, _NP), 2)
    padm = jnp.where(lane < _N, 0.0, -1e9).astype(f32)

    ones3 = jnp.ones((_BB, _NP, 1), f32)

    def gat(cin, W, WA, heads, dout):
        Wh = jnp.dot(cin, W, preferred_element_type=f32)
        sd = jnp.dot(cin, WA, preferred_element_type=f32)
        Wh3 = Wh.reshape(_BB, _NP, heads * dout)
        outs = []
        for hd in range(heads):
            src3 = sd[:, hd:hd + 1].reshape(_BB, _NP, 1)
            dst3 = jnp.transpose(
                sd[:, heads + hd:heads + hd + 1].reshape(_BB, _NP, 1),
                (0, 2, 1))
            t = src3 + jnp.max(dst3, axis=2, keepdims=True)
            rowmax = jnp.maximum(t, 0.2 * t)
            e = src3 + dst3
            e = jnp.maximum(e, 0.2 * e) + (padm - rowmax)
            w = jnp.exp(e)
            rhs = jnp.concatenate(
                [Wh3[:, :, hd * dout:(hd + 1) * dout], ones3], axis=2)
            nd = jax.lax.dot_general(
                w, rhs, (((2,), (1,)), ((0,), (0,))),
                preferred_element_type=f32)
            outs.append(nd[:, :, :dout] * (1.0 / nd[:, :, dout:dout + 1]))
        o = outs[0] if heads == 1 else jnp.concatenate(outs, axis=2)
        return o.reshape(_P, heads * dout)

    c1 = gat(cp, W_g1_ref[...], A1_ref[...], 4, 32)
    c1 = jnp.where(c1 > 0, c1, jnp.exp(jnp.minimum(c1, 0.0)) - 1.0)
    c2 = gat(c1, W_g2_ref[...], A2_ref[...], 1, 128)

    co = jnp.concatenate([c2[k * _NP:k * _NP + _N, :] for k in range(_BB)],
                         axis=0)

    comm_out = jnp.dot(co, W_md_ref[...], preferred_element_type=f32) + b_md_ref[...]
    feat = jnp.concatenate([h, comm_out], axis=1)
    out_ref[...] = (jnp.dot(feat, W_act_ref[...], preferred_element_type=f32)
                    + b_act_ref[...])


def kernel(obs, rnn_states, masks, W_obs, b_obs, W_ih, W_hh, b_ih, b_hh,
           W_me, b_me, W_g1, a_g1, W_g2, a_g2, W_md, b_md, W_act, b_act):
    BN = _B * _N
    rnn2 = rnn_states.reshape(BN, 2 * _H)
    b_gates = (b_ih + b_hh).reshape(1, 4 * _H)

    eye4 = jnp.eye(4, dtype=jnp.float32)
    A1s = (a_g1[:, :32, None] * eye4[:, None, :]).reshape(128, 4)
    A1d = (a_g1[:, 32:, None] * eye4[:, None, :]).reshape(128, 4)
    A1 = jnp.concatenate([A1s, A1d], axis=1)
    A2 = jnp.stack([a_g2[0, :128], a_g2[0, 128:]], axis=1)
    WA1 = W_g1 @ A1
    WA2 = W_g2 @ A2

    def full(shape):
        return pl.BlockSpec(shape, lambda i: (0,) * len(shape))

    grid = (_B // _BB,)
    return pl.pallas_call(
        _fused,
        grid=grid,
        in_specs=[
            pl.BlockSpec((_R, 128), lambda i: (i, 0)),
            pl.BlockSpec((_R, 2 * _H), lambda i: (i, 0)),
            pl.BlockSpec((_R, 1), lambda i: (i, 0)),
            full((128, _H)),
            full((1, _H)),
            full((_H, 4 * _H)),
            full((_H, 4 * _H)),
            full((1, 4 * _H)),
            full((_H, _H)),
            full((1, _H)),
            full((_H, 128)),
            full((128, 8)),
            full((128, 128)),
            full((128, 2)),
            full((_H, _H)),
            full((1, _H)),
            full((2 * _H, _ACT)),
            full((1, _ACT)),
        ],
        out_specs=pl.BlockSpec((_R, _ACT), lambda i: (i, 0)),
        out_shape=jax.ShapeDtypeStruct((BN, _ACT), jnp.float32),
        compiler_params=pltpu.CompilerParams(
            dimension_semantics=("arbitrary",)),
    )(obs, rnn2, masks, W_obs, b_obs.reshape(1, _H), W_ih, W_hh, b_gates,
      W_me, b_me.reshape(1, _H), W_g1, WA1, W_g2, WA2, W_md,
      b_md.reshape(1, _H), W_act, b_act.reshape(1, _ACT))


# speedup vs baseline: 1.0387x; 1.0387x over previous
import jax
import jax.numpy as jnp
from jax.experimental import pallas as pl
from jax.experimental.pallas import tpu as pltpu

_B, _N, _H = 1024, 50, 128
_ACT = 19
_BB = 32
_R = _BB * _N
_NP = 64
_P = _BB * _NP


def _fused(obs_ref, rnn_ref, masks_ref, W_obs_ref, b_obs_ref, W_ih_ref,
           W_hh_ref, b_gates_ref, W_me_ref, b_me_ref, W_g1_ref, A1_ref,
           W_g2_ref, A2_ref, W_md_ref, b_md_ref, W_act_ref, b_act_ref,
           out_ref):
    f32 = jnp.float32

    x = jnp.tanh(jnp.dot(obs_ref[...], W_obs_ref[...],
                         preferred_element_type=f32) + b_obs_ref[...])
    m = masks_ref[...]
    h0 = rnn_ref[:, :_H] * m
    c0 = rnn_ref[:, _H:] * m
    gates = (jnp.dot(x, W_ih_ref[...], preferred_element_type=f32)
             + jnp.dot(h0, W_hh_ref[...], preferred_element_type=f32)
             + b_gates_ref[...])
    ig = jax.nn.sigmoid(gates[:, :_H])
    fg = jax.nn.sigmoid(gates[:, _H:2 * _H])
    gg = jnp.tanh(gates[:, 2 * _H:3 * _H])
    og = jax.nn.sigmoid(gates[:, 3 * _H:])
    c = fg * c0 + ig * gg
    h = og * jnp.tanh(c)

    comm = jnp.dot(h, W_me_ref[...], preferred_element_type=f32) + b_me_ref[...]

    zpad = jnp.zeros((_NP - _N, _H), f32)
    pieces = []
    for k in range(_BB):
        pieces.append(comm[k * _N:(k + 1) * _N, :])
        pieces.append(zpad)
    cp = jnp.concatenate(pieces, axis=0)

    lane = jax.lax.broadcasted_iota(jnp.int32, (1, 1, _NP), 2)
    padm = jnp.where(lane < _N, 0.0, -1e9).astype(f32)


    def gat(cin, W, WA, heads, dout):
        Wh = jnp.dot(cin, W, preferred_element_type=f32)
        sd = jnp.dot(cin, WA, preferred_element_type=f32)
        Wh3 = Wh.reshape(_BB, _NP, heads * dout)
        outs = []
        for hd in range(heads):
            src3 = sd[:, hd:hd + 1].reshape(_BB, _NP, 1)
            dst3 = jnp.transpose(
                sd[:, heads + hd:heads + hd + 1].reshape(_BB, _NP, 1),
                (0, 2, 1))
            t = src3 + jnp.max(dst3, axis=2, keepdims=True)
            rowmax = jnp.maximum(t, 0.2 * t)
            e = src3 + dst3
            e = jnp.maximum(e, 0.2 * e) + (padm - rowmax)
            w = jnp.exp(e)
            num = jax.lax.dot_general(
                w, Wh3[:, :, hd * dout:(hd + 1) * dout],
                (((2,), (1,)), ((0,), (0,))),
                preferred_element_type=f32)
            outs.append(num / jnp.sum(w, axis=2, keepdims=True))
        o = outs[0] if heads == 1 else jnp.concatenate(outs, axis=2)
        return o.reshape(_P, heads * dout)

    c1 = gat(cp, W_g1_ref[...], A1_ref[...], 4, 32)
    c1 = jnp.where(c1 > 0, c1, jnp.exp(jnp.minimum(c1, 0.0)) - 1.0)
    c2 = gat(c1, W_g2_ref[...], A2_ref[...], 1, 128)

    co = jnp.concatenate([c2[k * _NP:k * _NP + _N, :] for k in range(_BB)],
                         axis=0)

    comm_out = jnp.dot(co, W_md_ref[...], preferred_element_type=f32) + b_md_ref[...]
    feat = jnp.concatenate([h, comm_out], axis=1)
    out_ref[...] = (jnp.dot(feat, W_act_ref[...], preferred_element_type=f32)
                    + b_act_ref[...])


def kernel(obs, rnn_states, masks, W_obs, b_obs, W_ih, W_hh, b_ih, b_hh,
           W_me, b_me, W_g1, a_g1, W_g2, a_g2, W_md, b_md, W_act, b_act):
    BN = _B * _N
    rnn2 = rnn_states.reshape(BN, 2 * _H)
    b_gates = (b_ih + b_hh).reshape(1, 4 * _H)

    eye4 = jnp.eye(4, dtype=jnp.float32)
    A1s = (a_g1[:, :32, None] * eye4[:, None, :]).reshape(128, 4)
    A1d = (a_g1[:, 32:, None] * eye4[:, None, :]).reshape(128, 4)
    A1 = jnp.concatenate([A1s, A1d], axis=1)
    A2 = jnp.stack([a_g2[0, :128], a_g2[0, 128:]], axis=1)
    WA1 = W_g1 @ A1
    WA2 = W_g2 @ A2

    def full(shape):
        return pl.BlockSpec(shape, lambda i: (0,) * len(shape))

    grid = (_B // _BB,)
    return pl.pallas_call(
        _fused,
        grid=grid,
        in_specs=[
            pl.BlockSpec((_R, 128), lambda i: (i, 0)),
            pl.BlockSpec((_R, 2 * _H), lambda i: (i, 0)),
            pl.BlockSpec((_R, 1), lambda i: (i, 0)),
            full((128, _H)),
            full((1, _H)),
            full((_H, 4 * _H)),
            full((_H, 4 * _H)),
            full((1, 4 * _H)),
            full((_H, _H)),
            full((1, _H)),
            full((_H, 128)),
            full((128, 8)),
            full((128, 128)),
            full((128, 2)),
            full((_H, _H)),
            full((1, _H)),
            full((2 * _H, _ACT)),
            full((1, _ACT)),
        ],
        out_specs=pl.BlockSpec((_R, _ACT), lambda i: (i, 0)),
        out_shape=jax.ShapeDtypeStruct((BN, _ACT), jnp.float32),
        compiler_params=pltpu.CompilerParams(
            dimension_semantics=("parallel",)),
    )(obs, rnn2, masks, W_obs, b_obs.reshape(1, _H), W_ih, W_hh, b_gates,
      W_me, b_me.reshape(1, _H), W_g1, WA1, W_g2, WA2, W_md,
      b_md.reshape(1, _H), W_act, b_act.reshape(1, _ACT))


# speedup vs baseline: 1.0996x; 1.0586x over previous
import jax
import jax.numpy as jnp
from jax.experimental import pallas as pl
from jax.experimental.pallas import tpu as pltpu

_B, _N, _H = 1024, 50, 128
_ACT = 19
_BB = 32
_R = _BB * _N
_NP = 64
_P = _BB * _NP


def _fused(obs_ref, rnn_ref, masks_ref, W_obs_ref, b_obs_ref, W_ih_ref,
           W_hh_ref, b_gates_ref, W_me_ref, b_me_ref, W_g1_ref, A1_ref,
           W_g2_ref, A2_ref, W_md_ref, b_md_ref, W_act_ref, b_act_ref,
           out_ref):
    f32 = jnp.float32

    x = jnp.tanh(jnp.dot(obs_ref[...], W_obs_ref[...],
                         preferred_element_type=f32) + b_obs_ref[...])
    m = masks_ref[...]
    h0 = rnn_ref[:, :_H] * m
    c0 = rnn_ref[:, _H:] * m
    gates = (jnp.dot(x, W_ih_ref[...], preferred_element_type=f32)
             + jnp.dot(h0, W_hh_ref[...], preferred_element_type=f32)
             + b_gates_ref[...])
    ig = jax.nn.sigmoid(gates[:, :_H])
    fg = jax.nn.sigmoid(gates[:, _H:2 * _H])
    gg = jnp.tanh(gates[:, 2 * _H:3 * _H])
    og = jax.nn.sigmoid(gates[:, 3 * _H:])
    c = fg * c0 + ig * gg
    h = og * jnp.tanh(c)

    comm = jnp.dot(h, W_me_ref[...], preferred_element_type=f32) + b_me_ref[...]

    zpad = jnp.zeros((_NP - _N, _H), f32)
    pieces = []
    for k in range(_BB):
        pieces.append(comm[k * _N:(k + 1) * _N, :])
        pieces.append(zpad)
    cp = jnp.concatenate(pieces, axis=0)

    lane = jax.lax.broadcasted_iota(jnp.int32, (1, 1, _NP), 2)
    padm = jnp.where(lane < _N, 0.0, -1e9).astype(f32)


    def gat(cin, W, WA, heads, dout):
        Wh = jnp.dot(cin, W, preferred_element_type=f32)
        sd = jnp.dot(cin, WA, preferred_element_type=f32)
        Wh3 = Wh.reshape(_BB, _NP, heads * dout)
        if heads == 1:
            src_s = sd[:, 0:1].reshape(_BB, _NP, 1)
            dst_s = sd[:, 1:2].reshape(_BB, _NP, 1)
            rhs_s = Wh3
        else:
            src_s = jnp.concatenate(
                [sd[:, hd:hd + 1].reshape(_BB, _NP, 1)
                 for hd in range(heads)], axis=0)
            dst_s = jnp.concatenate(
                [sd[:, heads + hd:heads + hd + 1].reshape(_BB, _NP, 1)
                 for hd in range(heads)], axis=0)
            rhs_s = jnp.concatenate(
                [Wh3[:, :, hd * dout:(hd + 1) * dout]
                 for hd in range(heads)], axis=0)
        dst_t = jnp.transpose(dst_s, (0, 2, 1))
        t = src_s + jnp.max(dst_t, axis=2, keepdims=True)
        rowmax = jnp.maximum(t, 0.2 * t)
        e = src_s + dst_t
        e = jnp.maximum(e, 0.2 * e) + (padm - rowmax)
        w = jnp.exp(e)
        num = jax.lax.dot_general(
            w, rhs_s, (((2,), (1,)), ((0,), (0,))),
            preferred_element_type=f32)
        o = num / jnp.sum(w, axis=2, keepdims=True)
        if heads > 1:
            o = jnp.concatenate(
                [o[hd * _BB:(hd + 1) * _BB] for hd in range(heads)], axis=2)
        return o.reshape(_P, heads * dout)

    c1 = gat(cp, W_g1_ref[...], A1_ref[...], 4, 32)
    c1 = jnp.where(c1 > 0, c1, jnp.exp(jnp.minimum(c1, 0.0)) - 1.0)
    c2 = gat(c1, W_g2_ref[...], A2_ref[...], 1, 128)

    co = jnp.concatenate([c2[k * _NP:k * _NP + _N, :] for k in range(_BB)],
                         axis=0)

    comm_out = jnp.dot(co, W_md_ref[...], preferred_element_type=f32) + b_md_ref[...]
    feat = jnp.concatenate([h, comm_out], axis=1)
    out_ref[...] = (jnp.dot(feat, W_act_ref[...], preferred_element_type=f32)
                    + b_act_ref[...])


def kernel(obs, rnn_states, masks, W_obs, b_obs, W_ih, W_hh, b_ih, b_hh,
           W_me, b_me, W_g1, a_g1, W_g2, a_g2, W_md, b_md, W_act, b_act):
    BN = _B * _N
    rnn2 = rnn_states.reshape(BN, 2 * _H)
    b_gates = (b_ih + b_hh).reshape(1, 4 * _H)

    eye4 = jnp.eye(4, dtype=jnp.float32)
    A1s = (a_g1[:, :32, None] * eye4[:, None, :]).reshape(128, 4)
    A1d = (a_g1[:, 32:, None] * eye4[:, None, :]).reshape(128, 4)
    A1 = jnp.concatenate([A1s, A1d], axis=1)
    A2 = jnp.stack([a_g2[0, :128], a_g2[0, 128:]], axis=1)
    WA1 = W_g1 @ A1
    WA2 = W_g2 @ A2

    def full(shape):
        return pl.BlockSpec(shape, lambda i: (0,) * len(shape))

    grid = (_B // _BB,)
    return pl.pallas_call(
        _fused,
        grid=grid,
        in_specs=[
            pl.BlockSpec((_R, 128), lambda i: (i, 0)),
            pl.BlockSpec((_R, 2 * _H), lambda i: (i, 0)),
            pl.BlockSpec((_R, 1), lambda i: (i, 0)),
            full((128, _H)),
            full((1, _H)),
            full((_H, 4 * _H)),
            full((_H, 4 * _H)),
            full((1, 4 * _H)),
            full((_H, _H)),
            full((1, _H)),
            full((_H, 128)),
            full((128, 8)),
            full((128, 128)),
            full((128, 2)),
            full((_H, _H)),
            full((1, _H)),
            full((2 * _H, _ACT)),
            full((1, _ACT)),
        ],
        out_specs=pl.BlockSpec((_R, _ACT), lambda i: (i, 0)),
        out_shape=jax.ShapeDtypeStruct((BN, _ACT), jnp.float32),
        compiler_params=pltpu.CompilerParams(
            dimension_semantics=("parallel",)),
    )(obs, rnn2, masks, W_obs, b_obs.reshape(1, _H), W_ih, W_hh, b_gates,
      W_me, b_me.reshape(1, _H), W_g1, WA1, W_g2, WA2, W_md,
      b_md.reshape(1, _H), W_act, b_act.reshape(1, _ACT))
